# baseline (speedup 1.0000x reference)
import functools

import jax
import jax.numpy as jnp
from jax import lax
from jax.experimental import pallas as pl
from jax.experimental.pallas import tpu as pltpu
from jax.experimental.pallas import tpu_sc as plsc

MAXLEN = 200
EMBED = 64
HALF = 100
LANES = 16


def kernel(inputs, token_table, pos_table):
    B, L = inputs.shape
    NC, NS = 2, 16
    NW = NC * NS
    seq_per_w = B // NW
    idx3 = inputs.reshape(B, 2, HALF).astype(jnp.int32)

    mesh = plsc.VectorSubcoreMesh(
        core_axis_name="c", subcore_axis_name="s", num_cores=NC, num_subcores=NS
    )

    @functools.partial(
        pl.kernel,
        out_type=jax.ShapeDtypeStruct((B * L, EMBED), jnp.float32),
        mesh=mesh,
        scratch_types=[
            pltpu.VMEM((seq_per_w, 2, HALF), jnp.int32),
            pltpu.VMEM((MAXLEN, EMBED), jnp.float32),
            pltpu.VMEM((MAXLEN, EMBED), jnp.float32),
            pltpu.SemaphoreType.DMA,
        ],
        compiler_params=pltpu.CompilerParams(use_tc_tiling_on_sc=False),
    )
    def run(idx_hbm, table_hbm, pos_hbm, out_hbm, idx_v, pos_v, rows_v, sem):
        wid = lax.axis_index("s") * NC + lax.axis_index("c")
        base = wid * seq_per_w
        pltpu.sync_copy(pos_hbm, pos_v)
        pltpu.sync_copy(idx_hbm.at[pl.ds(base, seq_per_w)], idx_v)

        def seq_body(s, carry):
            pltpu.async_copy(
                table_hbm.at[idx_v.at[s, 0]], rows_v.at[pl.ds(0, HALF)], sem
            ).wait()
            pltpu.async_copy(
                table_hbm.at[idx_v.at[s, 1]], rows_v.at[pl.ds(HALF, HALF)], sem
            ).wait()

            def add_body(i, c):
                for j in range(EMBED // LANES):
                    sl = pl.ds(j * LANES, LANES)
                    rows_v[i, sl] += pos_v[i, sl]
                return c

            lax.fori_loop(0, MAXLEN, add_body, 0)
            pltpu.sync_copy(
                rows_v, out_hbm.at[pl.ds((base + s) * MAXLEN, MAXLEN)]
            )
            return carry

        lax.fori_loop(0, seq_per_w, seq_body, 0)

    out = run(idx3, token_table, pos_table)
    return out.reshape(B, L, EMBED)


# speedup vs baseline: 1.2235x; 1.2235x over previous
import functools

import jax
import jax.numpy as jnp
from jax import lax
from jax.experimental import pallas as pl
from jax.experimental.pallas import tpu as pltpu
from jax.experimental.pallas import tpu_sc as plsc

MAXLEN = 200
EMBED = 64
HALF = 100
LANES = 16
NBUF = 4


def kernel(inputs, token_table, pos_table):
    B, L = inputs.shape
    NC, NS = 2, 16
    NW = NC * NS
    seq_per_w = B // NW
    idx3 = inputs.reshape(B, 2, HALF).astype(jnp.int32)

    mesh = plsc.VectorSubcoreMesh(
        core_axis_name="c", subcore_axis_name="s", num_cores=NC, num_subcores=NS
    )

    @functools.partial(
        pl.kernel,
        out_type=jax.ShapeDtypeStruct((B * L, EMBED), jnp.float32),
        mesh=mesh,
        scratch_types=[
            pltpu.VMEM((seq_per_w, 2, HALF), jnp.int32),
            pltpu.VMEM((MAXLEN, EMBED), jnp.float32),
            pltpu.VMEM((NBUF, MAXLEN, EMBED), jnp.float32),
            pltpu.SemaphoreType.DMA((NBUF,)),
            pltpu.SemaphoreType.DMA((NBUF,)),
        ],
        compiler_params=pltpu.CompilerParams(use_tc_tiling_on_sc=False),
    )
    def run(idx_hbm, table_hbm, pos_hbm, out_hbm, idx_v, pos_v, rows_v, gsem, ssem):
        wid = lax.axis_index("s") * NC + lax.axis_index("c")
        base = wid * seq_per_w
        pltpu.sync_copy(pos_hbm, pos_v)
        pltpu.sync_copy(idx_hbm.at[pl.ds(base, seq_per_w)], idx_v)

        def issue_gather(s, b):
            pltpu.async_copy(
                table_hbm.at[idx_v.at[s, 0]],
                rows_v.at[b, pl.ds(0, HALF)],
                gsem.at[b],
            )
            pltpu.async_copy(
                table_hbm.at[idx_v.at[s, 1]],
                rows_v.at[b, pl.ds(HALF, HALF)],
                gsem.at[b],
            )

        def wait_gather(s, b):
            pltpu.make_async_copy(
                table_hbm.at[idx_v.at[s, 0]],
                rows_v.at[b, pl.ds(0, HALF)],
                gsem.at[b],
            ).wait()
            pltpu.make_async_copy(
                table_hbm.at[idx_v.at[s, 1]],
                rows_v.at[b, pl.ds(HALF, HALF)],
                gsem.at[b],
            ).wait()

        def out_slice(s):
            return out_hbm.at[pl.ds((base + s) * MAXLEN, MAXLEN)]

        def wait_store(s, b):
            pltpu.make_async_copy(rows_v.at[b], out_slice(s), ssem.at[b]).wait()

        for b in range(NBUF - 1):
            issue_gather(b, b)

        def outer(i, carry):
            for b in range(NBUF):
                s = i * NBUF + b
                wait_gather(s, b)

                @plsc.parallel_loop(0, MAXLEN, unroll=2)
                def _(t):
                    for j in range(EMBED // LANES):
                        sl = pl.ds(j * LANES, LANES)
                        rows_v[b, t, sl] += pos_v[t, sl]

                pltpu.async_copy(rows_v.at[b], out_slice(s), ssem.at[b])

                bp = (b + NBUF - 1) % NBUF

                @pl.when((s >= 1) & (s + NBUF - 1 < seq_per_w))
                def _():
                    wait_store(s - 1, bp)

                @pl.when(s + NBUF - 1 < seq_per_w)
                def _():
                    issue_gather(s + NBUF - 1, bp)
            return carry

        lax.fori_loop(0, seq_per_w // NBUF, outer, 0)

        for b in range(NBUF):
            wait_store(seq_per_w - NBUF + b, b)

    out = run(idx3, token_table, pos_table)
    return out.reshape(B, L, EMBED)
